# baseline (speedup 1.0000x reference)
import functools

import jax
import jax.numpy as jnp
from jax import lax
from jax.experimental import pallas as pl
from jax.experimental.pallas import tpu as pltpu
from jax.experimental.pallas import tpu_sc as plsc

N = 10000
NP = 10240
NT = 16
SL = NP // NT
NV = SL // 16
E = 320000
EPT = E // NT
NEC = 10
EC = EPT // NEC
ECV = EC // 16
G = 64
GP = 128
D = 128
RPT = N // NT
XCH = 25
NXCH = RPT // XCH
F32MIN = -3.4e38
INTMIN = -2147483648


def _zeros16(dtype=jnp.float32):
    return jnp.zeros((16,), dtype)


def _sc_body(gi_hbm, g_hbm, src_hbm, dst_hbm, xf_hbm, params_hbm,
             rdsum_hbm, rdmax_hbm,
             gi_v, gcat_v, deg_v, dinv_v, u_v, score_v, amp_v, sel_v,
             tmp_v, tmp2_v, srcb_v, dstb_v, hist_v, cnt_v, k_v, starts_v,
             ends_v, params_v, xbuf_v, accs_v, accm_v, outb_v,
             stage_s, red_s):
    sid = lax.axis_index("s")
    iota = lax.iota(jnp.int32, 16)

    pltpu.sync_copy(gi_hbm, gi_v)
    pltpu.sync_copy(g_hbm, gcat_v)
    pltpu.sync_copy(params_hbm, params_v)

    def _init(i, _):
        idx = iota + i * 16
        live = jnp.where(idx < N, 1.0, 0.0)
        sel_v[pl.ds(i * 16, 16)] = live
        amp_v[pl.ds(i * 16, 16)] = live
        return 0
    lax.fori_loop(0, NP // 16, _init, 0)

    for i in range(GP // 16):
        hist_v[pl.ds(i * 16, 16)] = _zeros16()
    base = sid * SL

    def _hist(i, _):
        idx = gi_v[pl.ds(base + i * 16, 16)]
        plsc.addupdate_scatter(hist_v, [idx], jnp.ones((16,), jnp.float32),
                               mask=jnp.ones((16,), jnp.bool_))
        return 0
    lax.fori_loop(0, NV, _hist, 0)
    pltpu.sync_copy(hist_v, stage_s.at[pl.ds(sid * NP, GP)])
    plsc.subcore_barrier()
    for i in range(GP // 16):
        hist_v[pl.ds(i * 16, 16)] = _zeros16()

    def _hsum(s, _):
        pltpu.sync_copy(stage_s.at[pl.ds(s * NP, GP)], tmp_v.at[pl.ds(0, GP)])
        for i in range(GP // 16):
            hist_v[pl.ds(i * 16, 16)] = (hist_v[pl.ds(i * 16, 16)]
                                         + tmp_v[pl.ds(i * 16, 16)])
        return 0
    lax.fori_loop(0, NT, _hsum, 0)
    plsc.subcore_barrier()

    carry = jnp.zeros((), jnp.int32)
    for i in range(GP // 16):
        c = hist_v[pl.ds(i * 16, 16)].astype(jnp.int32)
        cnt_v[pl.ds(i * 16, 16)] = c
        cs = plsc.cumsum(c)
        st = cs - c + carry
        starts_v[pl.ds(i * 16, 16)] = st
        ends_v[pl.ds(i * 16, 16)] = st + c
        carry = carry + jnp.sum(c)

    def reduce_nodearray():
        pltpu.sync_copy(deg_v, stage_s.at[pl.ds(sid * NP, NP)])
        plsc.subcore_barrier()
        soff = sid * SL

        def _z(i, _):
            tmp_v[pl.ds(i * 16, 16)] = _zeros16()
            return 0
        lax.fori_loop(0, NV, _z, 0)

        def _rs(s, _):
            pltpu.sync_copy(stage_s.at[pl.ds(s * NP + soff, SL)], tmp2_v)

            def _add(i, _):
                tmp_v[pl.ds(i * 16, 16)] = (tmp_v[pl.ds(i * 16, 16)]
                                            + tmp2_v[pl.ds(i * 16, 16)])
                return 0
            lax.fori_loop(0, NV, _add, 0, unroll=4)
            return 0
        lax.fori_loop(0, NT, _rs, 0)
        pltpu.sync_copy(tmp_v, red_s.at[pl.ds(soff, SL)])
        plsc.subcore_barrier()
        pltpu.sync_copy(red_s, deg_v)

    def edge_pass(score_mode, masked):
        def _z(i, _):
            deg_v[pl.ds(i * 16, 16)] = _zeros16()
            return 0
        lax.fori_loop(0, NP // 16, _z, 0)
        ebase = sid * EPT

        def _chunk(c, _):
            off = ebase + c * EC
            pltpu.sync_copy(src_hbm.at[pl.ds(off, EC)], srcb_v)
            pltpu.sync_copy(dst_hbm.at[pl.ds(off, EC)], dstb_v)

            def _ev(j, _):
                si = srcb_v[pl.ds(j * 16, 16)]
                di = dstb_v[pl.ds(j * 16, 16)]
                if masked:
                    v = (plsc.load_gather(sel_v, [si])
                         * plsc.load_gather(sel_v, [di]))
                    if score_mode:
                        v = v * plsc.load_gather(u_v, [si])
                elif score_mode:
                    v = plsc.load_gather(u_v, [si])
                else:
                    v = jnp.ones((16,), jnp.float32)
                plsc.addupdate_scatter(deg_v, [di], v)
                return 0
            lax.fori_loop(0, ECV, _ev, 0, unroll=5)
            return 0
        lax.fori_loop(0, NEC, _chunk, 0)
        reduce_nodearray()

    for layer in range(2):
        masked = layer > 0
        edge_pass(False, masked)

        goff = layer * NP

        def _dinv_u(i, _):
            ds_ = pl.ds(i * 16, 16)
            d = deg_v[ds_] + sel_v[ds_]
            s = plsc.bitcast(d, jnp.int32)
            y = plsc.bitcast(jnp.int32(0x5F3759DF) - (s >> 1), jnp.float32)
            y = y * (1.5 - 0.5 * d * y * y)
            y = y * (1.5 - 0.5 * d * y * y)
            y = y * (1.5 - 0.5 * d * y * y)
            dv = jnp.where(d > 0, y, 0.0)
            dinv_v[ds_] = dv
            u_v[ds_] = gcat_v[pl.ds(goff + i * 16, 16)] * amp_v[ds_] * dv
            return 0
        lax.fori_loop(0, NP // 16, _dinv_u, 0, unroll=4)

        edge_pass(True, masked)

        bscal = params_v[pl.ds(0, 16)][layer]

        def _score(i, _):
            ds_ = pl.ds(i * 16, 16)
            dv = dinv_v[ds_]
            sc = (dv * deg_v[ds_] + sel_v[ds_] * u_v[ds_] * dv
                  + jnp.full((16,), bscal))
            score_v[ds_] = sc
            sb = plsc.bitcast(sc, jnp.int32)
            key = sb ^ ((sb >> 1) >> 30 & jnp.int32(0x7FFFFFFF))
            keyeff = jnp.where(sel_v[ds_] > 0, key,
                               jnp.full((16,), INTMIN, jnp.int32))
            u_v[ds_] = plsc.bitcast(keyeff, jnp.float32)
            return 0
        lax.fori_loop(0, NP // 16, _score, 0, unroll=4)

        for i in range(GP // 16):
            c = cnt_v[pl.ds(i * 16, 16)]
            k_v[pl.ds(i * 16, 16)] = jnp.maximum((c + 1) // 2, 1)

        def _blk(bi, _):
            i0 = base + bi * 16
            kiv = plsc.bitcast(u_v[pl.ds(i0, 16)], jnp.int32)
            giv = gi_v[pl.ds(i0, 16)]
            j0 = plsc.load_gather(starts_v, [giv])[0]
            j1 = plsc.load_gather(ends_v, [giv])[15]
            iidx = iota + i0

            def _inner(j, rank):
                jv = jnp.full((16,), j)
                kj = plsc.bitcast(plsc.load_gather(u_v, [jv]), jnp.int32)
                gj = plsc.load_gather(gi_v, [jv])
                better = (kj > kiv) | ((kj == kiv) & (jv < iidx))
                m = (gj == giv) & better
                return rank + jnp.where(m, 1, 0).astype(jnp.int32)
            rank = lax.fori_loop(j0, j1, _inner, _zeros16(jnp.int32))
            kv = plsc.load_gather(k_v, [giv])
            selb = sel_v[pl.ds(i0, 16)]
            tmp_v[pl.ds(bi * 16, 16)] = jnp.where(
                (selb > 0) & (rank < kv), 1.0, 0.0)
            return 0
        lax.fori_loop(0, NV, _blk, 0)

        pltpu.sync_copy(tmp_v, stage_s.at[pl.ds(sid * SL, SL)])
        plsc.subcore_barrier()
        pltpu.sync_copy(stage_s.at[pl.ds(0, NP)], sel_v)
        plsc.subcore_barrier()

        def _amp(i, _):
            ds_ = pl.ds(i * 16, 16)
            e = jnp.exp(2.0 * score_v[ds_])
            th = 1.0 - 2.0 / (e + 1.0)
            amp_v[ds_] = sel_v[ds_] * amp_v[ds_] * th
            return 0
        lax.fori_loop(0, NP // 16, _amp, 0, unroll=4)

        for i in range(GP // 16):
            c = cnt_v[pl.ds(i * 16, 16)]
            cnt_v[pl.ds(i * 16, 16)] = jnp.where(c > 0, (c + 1) // 2, 0)

    def _zacc(i, _):
        accs_v[pl.ds(i * 16, 16)] = _zeros16()
        accm_v[pl.ds(i * 16, 16)] = jnp.full((16,), F32MIN)
        return 0
    lax.fori_loop(0, (65 * D) // 16, _zacc, 0)
    rbase = sid * RPT

    def _xch(c, _):
        r0 = rbase + c * XCH
        pltpu.sync_copy(xf_hbm.at[pl.ds(r0 * D, XCH * D)], xbuf_v)

        def _row(r, _):
            nv = jnp.full((16,), r0 + r)
            slive = plsc.load_gather(sel_v, [nv])[0]

            @pl.when(slive > 0.0)
            def _():
                av = plsc.load_gather(amp_v, [nv])
                o = plsc.load_gather(gi_v, [nv])[0] * D
                for cc in range(8):
                    y = xbuf_v[pl.ds(r * D + cc * 16, 16)] * av
                    accs_v[pl.ds(o + cc * 16, 16)] = (
                        accs_v[pl.ds(o + cc * 16, 16)] + y)
                    accm_v[pl.ds(o + cc * 16, 16)] = jnp.maximum(
                        accm_v[pl.ds(o + cc * 16, 16)], y)
            return 0
        lax.fori_loop(0, XCH, _row, 0)
        return 0
    lax.fori_loop(0, NXCH, _xch, 0)

    gb = sid * 4
    ob = gb * D
    pltpu.sync_copy(accs_v, stage_s.at[pl.ds(sid * NP, 65 * D)])
    plsc.subcore_barrier()

    def _zo(i, _):
        outb_v[pl.ds(i * 16, 16)] = _zeros16()
        return 0
    lax.fori_loop(0, 32, _zo, 0)

    def _rsum(s, _):
        pltpu.sync_copy(stage_s.at[pl.ds(s * NP + ob, 4 * D)],
                        tmp_v.at[pl.ds(0, 4 * D)])

        def _a(i, _):
            outb_v[pl.ds(i * 16, 16)] = (outb_v[pl.ds(i * 16, 16)]
                                         + tmp_v[pl.ds(i * 16, 16)])
            return 0
        lax.fori_loop(0, 32, _a, 0)
        return 0
    lax.fori_loop(0, NT, _rsum, 0)
    for gg in range(4):
        cg = plsc.load_gather(cnt_v, [jnp.full((16,), gb + gg, jnp.int32)])
        cf = jnp.maximum(cg, 1).astype(jnp.float32)
        for cc in range(8):
            ds_ = pl.ds(gg * D + cc * 16, 16)
            outb_v[ds_] = outb_v[ds_] / cf
    pltpu.sync_copy(outb_v, rdsum_hbm.at[pl.ds(ob, 4 * D)])
    plsc.subcore_barrier()

    pltpu.sync_copy(accm_v, stage_s.at[pl.ds(sid * NP, 65 * D)])
    plsc.subcore_barrier()

    def _zm(i, _):
        outb_v[pl.ds(i * 16, 16)] = jnp.full((16,), F32MIN)
        return 0
    lax.fori_loop(0, 32, _zm, 0)

    def _rmax(s, _):
        pltpu.sync_copy(stage_s.at[pl.ds(s * NP + ob, 4 * D)],
                        tmp_v.at[pl.ds(0, 4 * D)])

        def _m(i, _):
            outb_v[pl.ds(i * 16, 16)] = jnp.maximum(
                outb_v[pl.ds(i * 16, 16)], tmp_v[pl.ds(i * 16, 16)])
            return 0
        lax.fori_loop(0, 32, _m, 0)
        return 0
    lax.fori_loop(0, NT, _rmax, 0)
    for gg in range(4):
        zf = plsc.load_gather(cnt_v, [jnp.full((16,), gb + gg, jnp.int32)])
        for cc in range(8):
            ds_ = pl.ds(gg * D + cc * 16, 16)
            outb_v[ds_] = jnp.where(zf > 0, outb_v[ds_], 0.0)
    pltpu.sync_copy(outb_v, rdmax_hbm.at[pl.ds(ob, 4 * D)])


_sc_kernel = functools.partial(
    pl.kernel,
    out_type=[jax.ShapeDtypeStruct((G * D,), jnp.float32),
              jax.ShapeDtypeStruct((G * D,), jnp.float32)],
    mesh=plsc.VectorSubcoreMesh(core_axis_name="c", subcore_axis_name="s",
                                num_cores=1),
    compiler_params=pltpu.CompilerParams(needs_layout_passes=False),
    scratch_types=[
        pltpu.VMEM((NP,), jnp.int32),
        pltpu.VMEM((2 * NP,), jnp.float32),
        pltpu.VMEM((NP,), jnp.float32),
        pltpu.VMEM((NP,), jnp.float32),
        pltpu.VMEM((NP,), jnp.float32),
        pltpu.VMEM((NP,), jnp.float32),
        pltpu.VMEM((NP,), jnp.float32),
        pltpu.VMEM((NP,), jnp.float32),
        pltpu.VMEM((SL,), jnp.float32),
        pltpu.VMEM((SL,), jnp.float32),
        pltpu.VMEM((EC,), jnp.int32),
        pltpu.VMEM((EC,), jnp.int32),
        pltpu.VMEM((GP,), jnp.float32),
        pltpu.VMEM((GP,), jnp.int32),
        pltpu.VMEM((GP,), jnp.int32),
        pltpu.VMEM((GP,), jnp.int32),
        pltpu.VMEM((GP,), jnp.int32),
        pltpu.VMEM((16,), jnp.float32),
        pltpu.VMEM((XCH * D,), jnp.float32),
        pltpu.VMEM((65 * D,), jnp.float32),
        pltpu.VMEM((65 * D,), jnp.float32),
        pltpu.VMEM((4 * D,), jnp.float32),
        pltpu.VMEM_SHARED((NT * NP,), jnp.float32),
        pltpu.VMEM_SHARED((NP,), jnp.float32),
    ],
)(_sc_body)


def _matvec_kernel(x_ref, w_ref, o_ref):
    o_ref[...] = jnp.dot(x_ref[...], w_ref[...],
                         preferred_element_type=jnp.float32)


def _final_linear_kernel(mean_ref, mx_ref, w0_ref, w1_ref, b_ref, o_ref):
    o_ref[...] = (jnp.dot(mean_ref[...], w0_ref[...],
                          preferred_element_type=jnp.float32)
                  + jnp.dot(mx_ref[...], w1_ref[...],
                            preferred_element_type=jnp.float32)
                  + b_ref[...])


def kernel(input_feature, edge_index, graph_indicator, attn_w0, attn_b0,
           attn_w1, attn_b1, lin_w, lin_b):
    x = input_feature
    wpad = jnp.pad(jnp.concatenate([attn_w0, attn_w1], axis=1),
                   ((0, 0), (0, 6)))
    g8 = pl.pallas_call(
        _matvec_kernel,
        out_shape=jax.ShapeDtypeStruct((N, 8), jnp.float32),
    )(x, wpad)
    gcat = jnp.concatenate([jnp.pad(g8[:, 0], (0, NP - N)),
                            jnp.pad(g8[:, 1], (0, NP - N))])
    gi_pad = jnp.concatenate([graph_indicator,
                              jnp.full((NP - N,), G, jnp.int32)])
    params = (jnp.zeros((16,), jnp.float32)
              .at[0].set(attn_b0[0]).at[1].set(attn_b1[0]))
    rdsum, rdmax = _sc_kernel(gi_pad, gcat, edge_index[0], edge_index[1],
                              x.reshape(-1), params)
    return pl.pallas_call(
        _final_linear_kernel,
        out_shape=jax.ShapeDtypeStruct((G, D), jnp.float32),
    )(rdsum.reshape(G, D), rdmax.reshape(G, D),
      lin_w[:D], lin_w[D:], lin_b[None, :])


# speedup vs baseline: 1.1289x; 1.1289x over previous
import functools

import jax
import jax.numpy as jnp
from jax import lax
from jax.experimental import pallas as pl
from jax.experimental.pallas import tpu as pltpu
from jax.experimental.pallas import tpu_sc as plsc

N = 10000
NP = 10240
NT = 16
SL = NP // NT
NV = SL // 16
E = 320000
EPT = E // NT
NEC = 10
EC = EPT // NEC
ECV = EC // 16
G = 64
GP = 128
D = 128
RPT = N // NT
XCH = 25
NXCH = RPT // XCH
F32MIN = -3.4e38
INTMIN = -2147483648


def _zeros16(dtype=jnp.float32):
    return jnp.zeros((16,), dtype)


def _sc_body(gi_hbm, g_hbm, src_hbm, dst_hbm, xf_hbm, params_hbm,
             rdsum_hbm, rdmax_hbm,
             gi_v, gcat_v, deg_v, dinv_v, u_v, score_v, amp_v, sel_v,
             tmp_v, tmp2_v, srcb_v, dstb_v, hist_v, cnt_v, k_v, starts_v,
             ends_v, params_v, xbuf_v, accs_v, accm_v, outb_v,
             stage_s, red_s, dsem):
    sid = lax.axis_index("s")
    iota = lax.iota(jnp.int32, 16)

    pltpu.sync_copy(gi_hbm, gi_v)
    pltpu.sync_copy(g_hbm, gcat_v)
    pltpu.sync_copy(params_hbm, params_v)

    def _init(i, _):
        idx = iota + i * 16
        live = jnp.where(idx < N, 1.0, 0.0)
        sel_v[pl.ds(i * 16, 16)] = live
        amp_v[pl.ds(i * 16, 16)] = live
        return 0
    lax.fori_loop(0, NP // 16, _init, 0)

    for i in range(GP // 16):
        hist_v[pl.ds(i * 16, 16)] = _zeros16()
    base = sid * SL

    def _hist(i, _):
        idx = gi_v[pl.ds(base + i * 16, 16)]
        plsc.addupdate_scatter(hist_v, [idx], jnp.ones((16,), jnp.float32),
                               mask=jnp.ones((16,), jnp.bool_))
        return 0
    lax.fori_loop(0, NV, _hist, 0)
    pltpu.sync_copy(hist_v, stage_s.at[pl.ds(sid * NP, GP)])
    plsc.subcore_barrier()
    for i in range(GP // 16):
        hist_v[pl.ds(i * 16, 16)] = _zeros16()

    def _hsum(s, _):
        pltpu.sync_copy(stage_s.at[pl.ds(s * NP, GP)], tmp_v.at[pl.ds(0, GP)])
        for i in range(GP // 16):
            hist_v[pl.ds(i * 16, 16)] = (hist_v[pl.ds(i * 16, 16)]
                                         + tmp_v[pl.ds(i * 16, 16)])
        return 0
    lax.fori_loop(0, NT, _hsum, 0)
    plsc.subcore_barrier()

    carry = jnp.zeros((), jnp.int32)
    for i in range(GP // 16):
        c = hist_v[pl.ds(i * 16, 16)].astype(jnp.int32)
        cnt_v[pl.ds(i * 16, 16)] = c
        cs = plsc.cumsum(c)
        st = cs - c + carry
        starts_v[pl.ds(i * 16, 16)] = st
        ends_v[pl.ds(i * 16, 16)] = st + c
        carry = carry + jnp.sum(c)

    def reduce_nodearray():
        pltpu.sync_copy(deg_v, stage_s.at[pl.ds(sid * NP, NP)])
        plsc.subcore_barrier()
        soff = sid * SL

        def _fire(s, _):
            pltpu.async_copy(stage_s.at[pl.ds(s * NP + soff, SL)],
                             score_v.at[pl.ds(s * SL, SL)], dsem)
            return 0
        lax.fori_loop(0, NT, _fire, 0)

        def _drain(s, _):
            pltpu.make_async_copy(stage_s.at[pl.ds(s * NP + soff, SL)],
                                  score_v.at[pl.ds(s * SL, SL)], dsem).wait()
            return 0
        lax.fori_loop(0, NT, _drain, 0)

        def _add(i, _):
            acc = score_v[pl.ds(i * 16, 16)]
            for s in range(1, NT):
                acc = acc + score_v[pl.ds(s * SL + i * 16, 16)]
            tmp_v[pl.ds(i * 16, 16)] = acc
            return 0
        lax.fori_loop(0, NV, _add, 0)
        pltpu.sync_copy(tmp_v, red_s.at[pl.ds(soff, SL)])
        plsc.subcore_barrier()
        pltpu.sync_copy(red_s, deg_v)

    def edge_pass(score_mode, masked):
        def _z(i, _):
            deg_v[pl.ds(i * 16, 16)] = _zeros16()
            return 0
        lax.fori_loop(0, NP // 16, _z, 0)
        ebase = sid * EPT

        def _chunk(c, _):
            off = ebase + c * EC
            pltpu.sync_copy(src_hbm.at[pl.ds(off, EC)], srcb_v)
            pltpu.sync_copy(dst_hbm.at[pl.ds(off, EC)], dstb_v)

            def _ev(j, _):
                si = srcb_v[pl.ds(j * 16, 16)]
                di = dstb_v[pl.ds(j * 16, 16)]
                if masked:
                    v = (plsc.load_gather(sel_v, [si])
                         * plsc.load_gather(sel_v, [di]))
                    if score_mode:
                        v = v * plsc.load_gather(u_v, [si])
                elif score_mode:
                    v = plsc.load_gather(u_v, [si])
                else:
                    v = jnp.ones((16,), jnp.float32)
                plsc.addupdate_scatter(deg_v, [di], v)
                return 0
            lax.fori_loop(0, ECV, _ev, 0)
            return 0
        lax.fori_loop(0, NEC, _chunk, 0)
        reduce_nodearray()

    for layer in range(2):
        masked = layer > 0
        edge_pass(False, masked)

        goff = layer * NP

        def _dinv_u(i, _):
            ds_ = pl.ds(i * 16, 16)
            d = deg_v[ds_] + sel_v[ds_]
            s = plsc.bitcast(d, jnp.int32)
            y = plsc.bitcast(jnp.int32(0x5F3759DF) - (s >> 1), jnp.float32)
            y = y * (1.5 - 0.5 * d * y * y)
            y = y * (1.5 - 0.5 * d * y * y)
            y = y * (1.5 - 0.5 * d * y * y)
            dv = jnp.where(d > 0, y, 0.0)
            dinv_v[ds_] = dv
            u_v[ds_] = gcat_v[pl.ds(goff + i * 16, 16)] * amp_v[ds_] * dv
            return 0
        lax.fori_loop(0, NP // 16, _dinv_u, 0)

        edge_pass(True, masked)

        bscal = params_v[pl.ds(0, 16)][layer]

        def _score(i, _):
            ds_ = pl.ds(i * 16, 16)
            dv = dinv_v[ds_]
            sc = (dv * deg_v[ds_] + sel_v[ds_] * u_v[ds_] * dv
                  + jnp.full((16,), bscal))
            score_v[ds_] = sc
            sb = plsc.bitcast(sc, jnp.int32)
            key = sb ^ ((sb >> 1) >> 30 & jnp.int32(0x7FFFFFFF))
            keyeff = jnp.where(sel_v[ds_] > 0, key,
                               jnp.full((16,), INTMIN, jnp.int32))
            u_v[ds_] = plsc.bitcast(keyeff, jnp.float32)
            return 0
        lax.fori_loop(0, NP // 16, _score, 0)

        for i in range(GP // 16):
            c = cnt_v[pl.ds(i * 16, 16)]
            k_v[pl.ds(i * 16, 16)] = jnp.maximum((c + 1) // 2, 1)

        def _blk(bi, _):
            i0 = base + bi * 16
            kiv = plsc.bitcast(u_v[pl.ds(i0, 16)], jnp.int32)
            giv = gi_v[pl.ds(i0, 16)]
            j0 = plsc.load_gather(starts_v, [giv])[0]
            j1 = plsc.load_gather(ends_v, [giv])[15]
            iidx = iota + i0

            def _inner(j, rank):
                jv = jnp.full((16,), j)
                kj = plsc.bitcast(plsc.load_gather(u_v, [jv]), jnp.int32)
                gj = plsc.load_gather(gi_v, [jv])
                better = (kj > kiv) | ((kj == kiv) & (jv < iidx))
                m = (gj == giv) & better
                return rank + jnp.where(m, 1, 0).astype(jnp.int32)
            rank = lax.fori_loop(j0, j1, _inner, _zeros16(jnp.int32))
            kv = plsc.load_gather(k_v, [giv])
            selb = sel_v[pl.ds(i0, 16)]
            tmp_v[pl.ds(bi * 16, 16)] = jnp.where(
                (selb > 0) & (rank < kv), 1.0, 0.0)
            return 0
        lax.fori_loop(0, NV, _blk, 0)

        pltpu.sync_copy(tmp_v, stage_s.at[pl.ds(sid * SL, SL)])
        plsc.subcore_barrier()
        pltpu.sync_copy(stage_s.at[pl.ds(0, NP)], sel_v)
        plsc.subcore_barrier()

        def _amp(i, _):
            ds_ = pl.ds(i * 16, 16)
            e = jnp.exp(2.0 * score_v[ds_])
            th = 1.0 - 2.0 / (e + 1.0)
            amp_v[ds_] = sel_v[ds_] * amp_v[ds_] * th
            return 0
        lax.fori_loop(0, NP // 16, _amp, 0)

        for i in range(GP // 16):
            c = cnt_v[pl.ds(i * 16, 16)]
            cnt_v[pl.ds(i * 16, 16)] = jnp.where(c > 0, (c + 1) // 2, 0)

    def _zacc(i, _):
        accs_v[pl.ds(i * 16, 16)] = _zeros16()
        accm_v[pl.ds(i * 16, 16)] = jnp.full((16,), F32MIN)
        return 0
    lax.fori_loop(0, (65 * D) // 16, _zacc, 0)
    rbase = sid * RPT

    def _xch(c, _):
        r0 = rbase + c * XCH
        pltpu.sync_copy(xf_hbm.at[pl.ds(r0 * D, XCH * D)], xbuf_v)

        def _row(r, _):
            nv = jnp.full((16,), r0 + r)
            slive = plsc.load_gather(sel_v, [nv])[0]

            @pl.when(slive > 0.0)
            def _():
                av = plsc.load_gather(amp_v, [nv])
                o = plsc.load_gather(gi_v, [nv])[0] * D
                for cc in range(8):
                    y = xbuf_v[pl.ds(r * D + cc * 16, 16)] * av
                    accs_v[pl.ds(o + cc * 16, 16)] = (
                        accs_v[pl.ds(o + cc * 16, 16)] + y)
                    accm_v[pl.ds(o + cc * 16, 16)] = jnp.maximum(
                        accm_v[pl.ds(o + cc * 16, 16)], y)
            return 0
        lax.fori_loop(0, XCH, _row, 0)
        return 0
    lax.fori_loop(0, NXCH, _xch, 0)

    gb = sid * 4
    ob = gb * D
    pltpu.sync_copy(accs_v, stage_s.at[pl.ds(sid * NP, 65 * D)])
    plsc.subcore_barrier()

    def _zo(i, _):
        outb_v[pl.ds(i * 16, 16)] = _zeros16()
        return 0
    lax.fori_loop(0, 32, _zo, 0)

    def _rsum(s, _):
        pltpu.sync_copy(stage_s.at[pl.ds(s * NP + ob, 4 * D)],
                        tmp_v.at[pl.ds(0, 4 * D)])

        def _a(i, _):
            outb_v[pl.ds(i * 16, 16)] = (outb_v[pl.ds(i * 16, 16)]
                                         + tmp_v[pl.ds(i * 16, 16)])
            return 0
        lax.fori_loop(0, 32, _a, 0)
        return 0
    lax.fori_loop(0, NT, _rsum, 0)
    for gg in range(4):
        cg = plsc.load_gather(cnt_v, [jnp.full((16,), gb + gg, jnp.int32)])
        cf = jnp.maximum(cg, 1).astype(jnp.float32)
        for cc in range(8):
            ds_ = pl.ds(gg * D + cc * 16, 16)
            outb_v[ds_] = outb_v[ds_] / cf
    pltpu.sync_copy(outb_v, rdsum_hbm.at[pl.ds(ob, 4 * D)])
    plsc.subcore_barrier()

    pltpu.sync_copy(accm_v, stage_s.at[pl.ds(sid * NP, 65 * D)])
    plsc.subcore_barrier()

    def _zm(i, _):
        outb_v[pl.ds(i * 16, 16)] = jnp.full((16,), F32MIN)
        return 0
    lax.fori_loop(0, 32, _zm, 0)

    def _rmax(s, _):
        pltpu.sync_copy(stage_s.at[pl.ds(s * NP + ob, 4 * D)],
                        tmp_v.at[pl.ds(0, 4 * D)])

        def _m(i, _):
            outb_v[pl.ds(i * 16, 16)] = jnp.maximum(
                outb_v[pl.ds(i * 16, 16)], tmp_v[pl.ds(i * 16, 16)])
            return 0
        lax.fori_loop(0, 32, _m, 0)
        return 0
    lax.fori_loop(0, NT, _rmax, 0)
    for gg in range(4):
        zf = plsc.load_gather(cnt_v, [jnp.full((16,), gb + gg, jnp.int32)])
        for cc in range(8):
            ds_ = pl.ds(gg * D + cc * 16, 16)
            outb_v[ds_] = jnp.where(zf > 0, outb_v[ds_], 0.0)
    pltpu.sync_copy(outb_v, rdmax_hbm.at[pl.ds(ob, 4 * D)])


_sc_kernel = functools.partial(
    pl.kernel,
    out_type=[jax.ShapeDtypeStruct((G * D,), jnp.float32),
              jax.ShapeDtypeStruct((G * D,), jnp.float32)],
    mesh=plsc.VectorSubcoreMesh(core_axis_name="c", subcore_axis_name="s",
                                num_cores=1),
    compiler_params=pltpu.CompilerParams(needs_layout_passes=False),
    scratch_types=[
        pltpu.VMEM((NP,), jnp.int32),
        pltpu.VMEM((2 * NP,), jnp.float32),
        pltpu.VMEM((NP,), jnp.float32),
        pltpu.VMEM((NP,), jnp.float32),
        pltpu.VMEM((NP,), jnp.float32),
        pltpu.VMEM((NP,), jnp.float32),
        pltpu.VMEM((NP,), jnp.float32),
        pltpu.VMEM((NP,), jnp.float32),
        pltpu.VMEM((SL,), jnp.float32),
        pltpu.VMEM((SL,), jnp.float32),
        pltpu.VMEM((EC,), jnp.int32),
        pltpu.VMEM((EC,), jnp.int32),
        pltpu.VMEM((GP,), jnp.float32),
        pltpu.VMEM((GP,), jnp.int32),
        pltpu.VMEM((GP,), jnp.int32),
        pltpu.VMEM((GP,), jnp.int32),
        pltpu.VMEM((GP,), jnp.int32),
        pltpu.VMEM((16,), jnp.float32),
        pltpu.VMEM((XCH * D,), jnp.float32),
        pltpu.VMEM((65 * D,), jnp.float32),
        pltpu.VMEM((65 * D,), jnp.float32),
        pltpu.VMEM((4 * D,), jnp.float32),
        pltpu.VMEM_SHARED((NT * NP,), jnp.float32),
        pltpu.VMEM_SHARED((NP,), jnp.float32),
        pltpu.SemaphoreType.DMA,
    ],
)(_sc_body)


def _matvec_kernel(x_ref, w_ref, o_ref):
    o_ref[...] = jnp.dot(x_ref[...], w_ref[...],
                         preferred_element_type=jnp.float32)


def _final_linear_kernel(mean_ref, mx_ref, w0_ref, w1_ref, b_ref, o_ref):
    o_ref[...] = (jnp.dot(mean_ref[...], w0_ref[...],
                          preferred_element_type=jnp.float32)
                  + jnp.dot(mx_ref[...], w1_ref[...],
                            preferred_element_type=jnp.float32)
                  + b_ref[...])


def kernel(input_feature, edge_index, graph_indicator, attn_w0, attn_b0,
           attn_w1, attn_b1, lin_w, lin_b):
    x = input_feature
    wpad = jnp.pad(jnp.concatenate([attn_w0, attn_w1], axis=1),
                   ((0, 0), (0, 6)))
    g8 = pl.pallas_call(
        _matvec_kernel,
        out_shape=jax.ShapeDtypeStruct((N, 8), jnp.float32),
    )(x, wpad)
    gcat = jnp.concatenate([jnp.pad(g8[:, 0], (0, NP - N)),
                            jnp.pad(g8[:, 1], (0, NP - N))])
    gi_pad = jnp.concatenate([graph_indicator,
                              jnp.full((NP - N,), G, jnp.int32)])
    params = (jnp.zeros((16,), jnp.float32)
              .at[0].set(attn_b0[0]).at[1].set(attn_b1[0]))
    rdsum, rdmax = _sc_kernel(gi_pad, gcat, edge_index[0], edge_index[1],
                              x.reshape(-1), params)
    return pl.pallas_call(
        _final_linear_kernel,
        out_shape=jax.ShapeDtypeStruct((G, D), jnp.float32),
    )(rdsum.reshape(G, D), rdmax.reshape(G, D),
      lin_w[:D], lin_w[D:], lin_b[None, :])


# speedup vs baseline: 1.1503x; 1.0189x over previous
import functools

import jax
import jax.numpy as jnp
from jax import lax
from jax.experimental import pallas as pl
from jax.experimental.pallas import tpu as pltpu
from jax.experimental.pallas import tpu_sc as plsc

N = 10000
NP = 10240
NT = 16
SL = NP // NT
NV = SL // 16
E = 320000
EPT = E // NT
NEC = 10
EC = EPT // NEC
ECV = EC // 16
G = 64
GP = 128
D = 128
RPT = N // NT
XCH = 25
NXCH = RPT // XCH
F32MIN = -3.4e38
INTMIN = -2147483648


def _zeros16(dtype=jnp.float32):
    return jnp.zeros((16,), dtype)


def _sc_body(gi_hbm, g_hbm, src_hbm, dst_hbm, xf_hbm, params_hbm,
             rdsum_hbm, rdmax_hbm,
             gi_v, gcat_v, deg_v, dinv_v, u_v, score_v, amp_v, sel_v,
             tmp_v, tmp2_v, srcb_v, dstb_v, hist_v, cnt_v, k_v, starts_v,
             ends_v, params_v, xbuf_v, accs_v, accm_v, outb_v,
             stage_s, red_s, dsem):
    sid = lax.axis_index("s")
    iota = lax.iota(jnp.int32, 16)

    pltpu.sync_copy(gi_hbm, gi_v)
    pltpu.sync_copy(g_hbm, gcat_v)
    pltpu.sync_copy(params_hbm, params_v)

    def _init(i, _):
        idx = iota + i * 16
        live = jnp.where(idx < N, 1.0, 0.0)
        sel_v[pl.ds(i * 16, 16)] = live
        amp_v[pl.ds(i * 16, 16)] = live
        return 0
    lax.fori_loop(0, NP // 16, _init, 0)

    for i in range(GP // 16):
        hist_v[pl.ds(i * 16, 16)] = _zeros16()
    base = sid * SL

    def _hist(i, _):
        idx = gi_v[pl.ds(base + i * 16, 16)]
        plsc.addupdate_scatter(hist_v, [idx], jnp.ones((16,), jnp.float32),
                               mask=jnp.ones((16,), jnp.bool_))
        return 0
    lax.fori_loop(0, NV, _hist, 0)
    pltpu.sync_copy(hist_v, stage_s.at[pl.ds(sid * NP, GP)])
    plsc.subcore_barrier()
    for i in range(GP // 16):
        hist_v[pl.ds(i * 16, 16)] = _zeros16()

    def _hsum(s, _):
        pltpu.sync_copy(stage_s.at[pl.ds(s * NP, GP)], tmp_v.at[pl.ds(0, GP)])
        for i in range(GP // 16):
            hist_v[pl.ds(i * 16, 16)] = (hist_v[pl.ds(i * 16, 16)]
                                         + tmp_v[pl.ds(i * 16, 16)])
        return 0
    lax.fori_loop(0, NT, _hsum, 0)
    plsc.subcore_barrier()

    carry = jnp.zeros((), jnp.int32)
    for i in range(GP // 16):
        c = hist_v[pl.ds(i * 16, 16)].astype(jnp.int32)
        cnt_v[pl.ds(i * 16, 16)] = c
        cs = plsc.cumsum(c)
        st = cs - c + carry
        starts_v[pl.ds(i * 16, 16)] = st
        ends_v[pl.ds(i * 16, 16)] = st + c
        carry = carry + jnp.sum(c)

    def reduce_nodearray():
        pltpu.sync_copy(deg_v, stage_s.at[pl.ds(sid * NP, NP)])
        plsc.subcore_barrier()
        soff = sid * SL

        def _fire(s, _):
            pltpu.async_copy(stage_s.at[pl.ds(s * NP + soff, SL)],
                             score_v.at[pl.ds(s * SL, SL)], dsem)
            return 0
        lax.fori_loop(0, NT, _fire, 0)

        def _drain(s, _):
            pltpu.make_async_copy(stage_s.at[pl.ds(s * NP + soff, SL)],
                                  score_v.at[pl.ds(s * SL, SL)], dsem).wait()
            return 0
        lax.fori_loop(0, NT, _drain, 0)

        def _add(i, _):
            acc = score_v[pl.ds(i * 16, 16)]
            for s in range(1, NT):
                acc = acc + score_v[pl.ds(s * SL + i * 16, 16)]
            tmp_v[pl.ds(i * 16, 16)] = acc
            return 0
        lax.fori_loop(0, NV, _add, 0)
        pltpu.sync_copy(tmp_v, red_s.at[pl.ds(soff, SL)])
        plsc.subcore_barrier()
        pltpu.sync_copy(red_s, deg_v)

    def edge_pass(score_mode, masked):
        def _z(i, _):
            deg_v[pl.ds(i * 16, 16)] = _zeros16()
            return 0
        lax.fori_loop(0, NP // 16, _z, 0)
        ebase = sid * EPT

        def _chunk(c, _):
            off = ebase + c * EC
            pltpu.sync_copy(src_hbm.at[pl.ds(off, EC)], srcb_v)
            pltpu.sync_copy(dst_hbm.at[pl.ds(off, EC)], dstb_v)

            def _ev(j, _):
                si = srcb_v[pl.ds(j * 16, 16)]
                di = dstb_v[pl.ds(j * 16, 16)]
                if masked:
                    v = (plsc.load_gather(sel_v, [si])
                         * plsc.load_gather(sel_v, [di]))
                    if score_mode:
                        v = v * plsc.load_gather(u_v, [si])
                elif score_mode:
                    v = plsc.load_gather(u_v, [si])
                else:
                    v = jnp.ones((16,), jnp.float32)
                plsc.addupdate_scatter(deg_v, [di], v)
                return 0
            lax.fori_loop(0, ECV, _ev, 0)
            return 0
        lax.fori_loop(0, NEC, _chunk, 0)
        reduce_nodearray()

    for layer in range(2):
        masked = layer > 0
        edge_pass(False, masked)

        goff = layer * NP

        def _dinv_u(i, _):
            ds_ = pl.ds(i * 16, 16)
            d = deg_v[ds_] + sel_v[ds_]
            s = plsc.bitcast(d, jnp.int32)
            y = plsc.bitcast(jnp.int32(0x5F3759DF) - (s >> 1), jnp.float32)
            y = y * (1.5 - 0.5 * d * y * y)
            y = y * (1.5 - 0.5 * d * y * y)
            y = y * (1.5 - 0.5 * d * y * y)
            dv = jnp.where(d > 0, y, 0.0)
            dinv_v[ds_] = dv
            u_v[ds_] = gcat_v[pl.ds(goff + i * 16, 16)] * amp_v[ds_] * dv
            return 0
        lax.fori_loop(0, NP // 16, _dinv_u, 0)

        edge_pass(True, masked)

        bscal = params_v[pl.ds(0, 16)][layer]

        def _score(i, _):
            ds_ = pl.ds(i * 16, 16)
            dv = dinv_v[ds_]
            sc = (dv * deg_v[ds_] + sel_v[ds_] * u_v[ds_] * dv
                  + jnp.full((16,), bscal))
            score_v[ds_] = sc
            sb = plsc.bitcast(sc, jnp.int32)
            key = sb ^ ((sb >> 1) >> 30 & jnp.int32(0x7FFFFFFF))
            keyeff = jnp.where(sel_v[ds_] > 0, key,
                               jnp.full((16,), INTMIN, jnp.int32))
            u_v[ds_] = plsc.bitcast(keyeff, jnp.float32)
            return 0
        lax.fori_loop(0, NP // 16, _score, 0)

        for i in range(GP // 16):
            c = cnt_v[pl.ds(i * 16, 16)]
            k_v[pl.ds(i * 16, 16)] = jnp.maximum((c + 1) // 2, 1)

        def _blk(bi, _):
            i0 = base + bi * 16
            kiv = plsc.bitcast(u_v[pl.ds(i0, 16)], jnp.int32)
            giv = gi_v[pl.ds(i0, 16)]
            j0 = plsc.load_gather(starts_v, [giv])[0]
            j1 = plsc.load_gather(ends_v, [giv])[15]
            iidx = iota + i0

            def _inner(j, rank):
                jv = jnp.full((16,), j)
                kj = plsc.bitcast(plsc.load_gather(u_v, [jv]), jnp.int32)
                gj = plsc.load_gather(gi_v, [jv])
                better = (kj > kiv) | ((kj == kiv) & (jv < iidx))
                m = (gj == giv) & better
                return rank + jnp.where(m, 1, 0).astype(jnp.int32)
            rank = lax.fori_loop(j0, j1, _inner, _zeros16(jnp.int32))
            kv = plsc.load_gather(k_v, [giv])
            selb = sel_v[pl.ds(i0, 16)]
            tmp_v[pl.ds(bi * 16, 16)] = jnp.where(
                (selb > 0) & (rank < kv), 1.0, 0.0)
            return 0
        lax.fori_loop(0, NV, _blk, 0)

        pltpu.sync_copy(tmp_v, stage_s.at[pl.ds(sid * SL, SL)])
        plsc.subcore_barrier()
        pltpu.sync_copy(stage_s.at[pl.ds(0, NP)], sel_v)
        plsc.subcore_barrier()

        def _amp(i, _):
            ds_ = pl.ds(i * 16, 16)
            e = jnp.exp(2.0 * score_v[ds_])
            th = 1.0 - 2.0 / (e + 1.0)
            amp_v[ds_] = sel_v[ds_] * amp_v[ds_] * th
            return 0
        lax.fori_loop(0, NP // 16, _amp, 0)

        for i in range(GP // 16):
            c = cnt_v[pl.ds(i * 16, 16)]
            cnt_v[pl.ds(i * 16, 16)] = jnp.where(c > 0, (c + 1) // 2, 0)

    def _zacc(i, _):
        accs_v[pl.ds(i * 16, 16)] = _zeros16()
        accm_v[pl.ds(i * 16, 16)] = jnp.full((16,), F32MIN)
        return 0
    lax.fori_loop(0, (65 * D) // 16, _zacc, 0)
    rbase = sid * RPT

    def _xch(c, _):
        r0 = rbase + c * XCH
        pltpu.sync_copy(xf_hbm.at[pl.ds(r0 * D, XCH * D)], xbuf_v)

        def _row(r, _):
            nv = jnp.full((16,), r0 + r)
            slive = plsc.load_gather(sel_v, [nv])[0]

            @pl.when(slive > 0.0)
            def _():
                av = plsc.load_gather(amp_v, [nv])
                o = plsc.load_gather(gi_v, [nv])[0] * D
                for cc in range(8):
                    y = xbuf_v[pl.ds(r * D + cc * 16, 16)] * av
                    accs_v[pl.ds(o + cc * 16, 16)] = (
                        accs_v[pl.ds(o + cc * 16, 16)] + y)
                    accm_v[pl.ds(o + cc * 16, 16)] = jnp.maximum(
                        accm_v[pl.ds(o + cc * 16, 16)], y)
            return 0
        lax.fori_loop(0, XCH, _row, 0)
        return 0
    lax.fori_loop(0, NXCH, _xch, 0)

    gb = sid * 4
    ob = gb * D
    pltpu.sync_copy(accs_v, stage_s.at[pl.ds(sid * NP, 65 * D)])
    plsc.subcore_barrier()

    def _fsum(s, _):
        pltpu.async_copy(stage_s.at[pl.ds(s * NP + ob, 4 * D)],
                         accs_v.at[pl.ds(s * 4 * D, 4 * D)], dsem)
        return 0
    lax.fori_loop(0, NT, _fsum, 0)

    def _dsum(s, _):
        pltpu.make_async_copy(stage_s.at[pl.ds(s * NP + ob, 4 * D)],
                              accs_v.at[pl.ds(s * 4 * D, 4 * D)],
                              dsem).wait()
        return 0
    lax.fori_loop(0, NT, _dsum, 0)

    def _rsum(i, _):
        acc = accs_v[pl.ds(i * 16, 16)]
        for s in range(1, NT):
            acc = acc + accs_v[pl.ds(s * 4 * D + i * 16, 16)]
        outb_v[pl.ds(i * 16, 16)] = acc
        return 0
    lax.fori_loop(0, 32, _rsum, 0)
    for gg in range(4):
        cg = plsc.load_gather(cnt_v, [jnp.full((16,), gb + gg, jnp.int32)])
        cf = jnp.maximum(cg, 1).astype(jnp.float32)
        for cc in range(8):
            ds_ = pl.ds(gg * D + cc * 16, 16)
            outb_v[ds_] = outb_v[ds_] / cf
    pltpu.sync_copy(outb_v, rdsum_hbm.at[pl.ds(ob, 4 * D)])
    plsc.subcore_barrier()

    pltpu.sync_copy(accm_v, stage_s.at[pl.ds(sid * NP, 65 * D)])
    plsc.subcore_barrier()

    def _fmax(s, _):
        pltpu.async_copy(stage_s.at[pl.ds(s * NP + ob, 4 * D)],
                         accm_v.at[pl.ds(s * 4 * D, 4 * D)], dsem)
        return 0
    lax.fori_loop(0, NT, _fmax, 0)

    def _dmax(s, _):
        pltpu.make_async_copy(stage_s.at[pl.ds(s * NP + ob, 4 * D)],
                              accm_v.at[pl.ds(s * 4 * D, 4 * D)],
                              dsem).wait()
        return 0
    lax.fori_loop(0, NT, _dmax, 0)

    def _rmax(i, _):
        acc = accm_v[pl.ds(i * 16, 16)]
        for s in range(1, NT):
            acc = jnp.maximum(acc, accm_v[pl.ds(s * 4 * D + i * 16, 16)])
        outb_v[pl.ds(i * 16, 16)] = acc
        return 0
    lax.fori_loop(0, 32, _rmax, 0)
    for gg in range(4):
        zf = plsc.load_gather(cnt_v, [jnp.full((16,), gb + gg, jnp.int32)])
        for cc in range(8):
            ds_ = pl.ds(gg * D + cc * 16, 16)
            outb_v[ds_] = jnp.where(zf > 0, outb_v[ds_], 0.0)
    pltpu.sync_copy(outb_v, rdmax_hbm.at[pl.ds(ob, 4 * D)])


_sc_kernel = functools.partial(
    pl.kernel,
    out_type=[jax.ShapeDtypeStruct((G * D,), jnp.float32),
              jax.ShapeDtypeStruct((G * D,), jnp.float32)],
    mesh=plsc.VectorSubcoreMesh(core_axis_name="c", subcore_axis_name="s",
                                num_cores=1),
    compiler_params=pltpu.CompilerParams(needs_layout_passes=False),
    scratch_types=[
        pltpu.VMEM((NP,), jnp.int32),
        pltpu.VMEM((2 * NP,), jnp.float32),
        pltpu.VMEM((NP,), jnp.float32),
        pltpu.VMEM((NP,), jnp.float32),
        pltpu.VMEM((NP,), jnp.float32),
        pltpu.VMEM((NP,), jnp.float32),
        pltpu.VMEM((NP,), jnp.float32),
        pltpu.VMEM((NP,), jnp.float32),
        pltpu.VMEM((SL,), jnp.float32),
        pltpu.VMEM((SL,), jnp.float32),
        pltpu.VMEM((EC,), jnp.int32),
        pltpu.VMEM((EC,), jnp.int32),
        pltpu.VMEM((GP,), jnp.float32),
        pltpu.VMEM((GP,), jnp.int32),
        pltpu.VMEM((GP,), jnp.int32),
        pltpu.VMEM((GP,), jnp.int32),
        pltpu.VMEM((GP,), jnp.int32),
        pltpu.VMEM((16,), jnp.float32),
        pltpu.VMEM((XCH * D,), jnp.float32),
        pltpu.VMEM((65 * D,), jnp.float32),
        pltpu.VMEM((65 * D,), jnp.float32),
        pltpu.VMEM((4 * D,), jnp.float32),
        pltpu.VMEM_SHARED((NT * NP,), jnp.float32),
        pltpu.VMEM_SHARED((NP,), jnp.float32),
        pltpu.SemaphoreType.DMA,
    ],
)(_sc_body)


def _matvec_kernel(x_ref, w_ref, o_ref):
    o_ref[...] = jnp.dot(x_ref[...], w_ref[...],
                         preferred_element_type=jnp.float32)


def _final_linear_kernel(mean_ref, mx_ref, w0_ref, w1_ref, b_ref, o_ref):
    o_ref[...] = (jnp.dot(mean_ref[...], w0_ref[...],
                          preferred_element_type=jnp.float32)
                  + jnp.dot(mx_ref[...], w1_ref[...],
                            preferred_element_type=jnp.float32)
                  + b_ref[...])


def kernel(input_feature, edge_index, graph_indicator, attn_w0, attn_b0,
           attn_w1, attn_b1, lin_w, lin_b):
    x = input_feature
    wpad = jnp.pad(jnp.concatenate([attn_w0, attn_w1], axis=1),
                   ((0, 0), (0, 6)))
    g8 = pl.pallas_call(
        _matvec_kernel,
        out_shape=jax.ShapeDtypeStruct((N, 8), jnp.float32),
    )(x, wpad)
    gcat = jnp.concatenate([jnp.pad(g8[:, 0], (0, NP - N)),
                            jnp.pad(g8[:, 1], (0, NP - N))])
    gi_pad = jnp.concatenate([graph_indicator,
                              jnp.full((NP - N,), G, jnp.int32)])
    params = (jnp.zeros((16,), jnp.float32)
              .at[0].set(attn_b0[0]).at[1].set(attn_b1[0]))
    rdsum, rdmax = _sc_kernel(gi_pad, gcat, edge_index[0], edge_index[1],
                              x.reshape(-1), params)
    return pl.pallas_call(
        _final_linear_kernel,
        out_shape=jax.ShapeDtypeStruct((G, D), jnp.float32),
    )(rdsum.reshape(G, D), rdmax.reshape(G, D),
      lin_w[:D], lin_w[D:], lin_b[None, :])
